# Initial kernel scaffold; baseline (speedup 1.0000x reference)
import jax
import jax.numpy as jnp
from jax.experimental import pallas as pl


def kernel(x, state, hyperedge_index, hyperedge_weight, gate_W, gate_b, gate_g, gate_beta, cand_W, cand_b, cand_g, cand_beta, res_W, res_b):
    raise NotImplementedError("write your pallas kernel here")


# speedup vs baseline: 8.1074x; 8.1074x over previous
import functools

import jax
import jax.numpy as jnp
from jax import lax
from jax.experimental import pallas as pl
from jax.experimental.pallas import tpu as pltpu
from jax.experimental.pallas import tpu_sc as plsc

N_NODES = 10000
NNZ = 320000
IN_F = 128
HID = 128

NP = 10240
W = 32
TPS = 16
CHUNK = 128
NNZ_PAD = 323584
NCH = NNZ_PAD // (TPS * CHUNK)
RPT = NP // TPS
PAD_BASE = 10000

_mesh = plsc.VectorSubcoreMesh(core_axis_name="c", subcore_axis_name="s")
_sc_params = pltpu.CompilerParams(use_tc_tiling_on_sc=False)


def _zero_vec(ref, n):
    z = jnp.zeros((16,), jnp.float32)

    def body(i, _):
        ref[pl.ds(i * 16, 16)] = z
        return 0

    lax.fori_loop(0, n // 16, body, 0)


def _fill_zbuf(zbuf):
    z = jnp.zeros((16,), jnp.float32)
    for r in range(16):
        for c in range(W // 16):
            zbuf[r, pl.ds(c * 16, 16)] = z


def _zero_shared_rows(zbuf, sh, base):
    def body(b, _):
        pltpu.sync_copy(zbuf, sh.at[pl.ds(base + b * 16, 16), :])
        return 0

    lax.fori_loop(0, RPT // 16, body, 0)


def _pair_sweep(nidx, eidx, src, dst, rbuf0, rbuf1, sem0, sem1):
    def body(j2, _):
        j = j2 * 2
        h0 = pltpu.async_copy(src.at[nidx.at[j]], rbuf0, sem0)
        h1 = pltpu.async_copy(src.at[nidx.at[j + 1]], rbuf1, sem1)
        h0.wait()
        pltpu.sync_copy(rbuf0, dst.at[eidx.at[j]], add=True)
        h1.wait()
        pltpu.sync_copy(rbuf1, dst.at[eidx.at[j + 1]], add=True)
        return 0

    lax.fori_loop(0, NCH // 2, body, 0)


def _scale_rows(ef_sh, base, bl, sbuf):
    def body(b, _):
        pltpu.sync_copy(ef_sh.at[pl.ds(base + b * 16, 16), :], sbuf)
        bv16 = bl[pl.ds(b * 16, 16)]
        for r in range(16):
            bvec = lax.broadcast(bv16[r], (16,))
            for c in range(W // 16):
                sbuf[r, pl.ds(c * 16, 16)] = sbuf[r, pl.ds(c * 16, 16)] * bvec
        pltpu.sync_copy(sbuf, ef_sh.at[pl.ds(base + b * 16, 16), :])
        return 0

    lax.fori_loop(0, RPT // 16, body, 0)


def _conv_slice(nidx, eidx, tbl, out_hbm, base, bl, sbuf, zbuf,
                rbuf0, rbuf1, sem0, sem1, ef_sh, out_sh):
    _zero_shared_rows(zbuf, ef_sh, base)
    _zero_shared_rows(zbuf, out_sh, base)
    plsc.subcore_barrier()
    _pair_sweep(nidx, eidx, tbl, ef_sh, rbuf0, rbuf1, sem0, sem1)
    plsc.subcore_barrier()
    _scale_rows(ef_sh, base, bl, sbuf)
    plsc.subcore_barrier()
    _pair_sweep(eidx, nidx, ef_sh, out_sh, rbuf0, rbuf1, sem0, sem1)
    plsc.subcore_barrier()
    pltpu.sync_copy(out_sh.at[pl.ds(base, RPT), :],
                    out_hbm.at[pl.ds(base, RPT), :])
    plsc.subcore_barrier()


_SC_SCRATCH = [
    pltpu.VMEM((NCH, CHUNK), jnp.int32),
    pltpu.VMEM((NCH, CHUNK), jnp.int32),
    pltpu.VMEM((CHUNK, W), jnp.float32),
    pltpu.VMEM((CHUNK, W), jnp.float32),
    pltpu.VMEM((16, W), jnp.float32),
    pltpu.VMEM((16, W), jnp.float32),
    pltpu.VMEM((RPT,), jnp.float32),
    pltpu.VMEM_SHARED((NP, W), jnp.float32),
    pltpu.VMEM_SHARED((NP, W), jnp.float32),
    pltpu.SemaphoreType.DMA,
    pltpu.SemaphoreType.DMA,
]


@functools.partial(
    pl.kernel,
    mesh=_mesh,
    compiler_params=_sc_params,
    out_type=(
        [jax.ShapeDtypeStruct((NP, W), jnp.float32)] * 8
        + [jax.ShapeDtypeStruct((NP,), jnp.float32)] * 2
    ),
    scratch_types=_SC_SCRATCH + [
        pltpu.VMEM((CHUNK,), jnp.float32),
        pltpu.VMEM((CHUNK,), jnp.float32),
        pltpu.VMEM((CHUNK,), jnp.float32),
        pltpu.VMEM((RPT,), jnp.float32),
        pltpu.VMEM_SHARED((NP,), jnp.float32),
        pltpu.VMEM_SHARED((NP,), jnp.float32),
    ],
)
def _gate_conv(xw0, xw1, xw2, xw3, xw4, xw5, xw6, xw7, nidx_h, eidx_h, hw_h,
               o0, o1, o2, o3, o4, o5, o6, o7, dinv_h, binv_h,
               nidx, eidx, rbuf0, rbuf1, sbuf, zbuf, bl, ef_sh, out_sh,
               sem0, sem1, hwbuf0, hwbuf1, ones, dl, d_sh, bd_sh):
    cid = lax.axis_index("c")
    sid = lax.axis_index("s")
    base = sid * RPT

    pltpu.sync_copy(nidx_h.at[sid], nidx)
    pltpu.sync_copy(eidx_h.at[sid], eidx)

    _fill_zbuf(zbuf)
    one = jnp.ones((16,), jnp.float32)
    for i in range(CHUNK // 16):
        ones[pl.ds(i * 16, 16)] = one

    _zero_vec(bl, RPT)
    pltpu.sync_copy(bl, d_sh.at[pl.ds(base, RPT)])
    pltpu.sync_copy(bl, bd_sh.at[pl.ds(base, RPT)])
    plsc.subcore_barrier()

    def hist(j2, _):
        j = j2 * 2
        h0 = pltpu.async_copy(hw_h.at[eidx.at[j]], hwbuf0, sem0)
        h1 = pltpu.async_copy(hw_h.at[eidx.at[j + 1]], hwbuf1, sem1)
        h0.wait()
        pltpu.sync_copy(hwbuf0, d_sh.at[nidx.at[j]], add=True)
        pltpu.sync_copy(ones, bd_sh.at[eidx.at[j]], add=True)
        h1.wait()
        pltpu.sync_copy(hwbuf1, d_sh.at[nidx.at[j + 1]], add=True)
        pltpu.sync_copy(ones, bd_sh.at[eidx.at[j + 1]], add=True)
        return 0

    lax.fori_loop(0, NCH // 2, hist, 0)
    plsc.subcore_barrier()

    pltpu.sync_copy(bd_sh.at[pl.ds(base, RPT)], bl)
    pltpu.sync_copy(d_sh.at[pl.ds(base, RPT)], dl)

    def recip(i, _):
        bd = bl[pl.ds(i * 16, 16)]
        bl[pl.ds(i * 16, 16)] = jnp.where(bd > 0.0, 1.0 / bd, 0.0)
        dv = dl[pl.ds(i * 16, 16)]
        dl[pl.ds(i * 16, 16)] = jnp.where(dv > 0.0, 1.0 / dv, 0.0)
        return 0

    lax.fori_loop(0, RPT // 16, recip, 0)

    @pl.when(cid == 0)
    def _():
        pltpu.sync_copy(bl, binv_h.at[pl.ds(base, RPT)])
        pltpu.sync_copy(dl, dinv_h.at[pl.ds(base, RPT)])

    lo = [(xw0, o0), (xw1, o1), (xw2, o2), (xw3, o3)]
    hi = [(xw4, o4), (xw5, o5), (xw6, o6), (xw7, o7)]
    for s in range(4):
        @pl.when(cid == 0)
        def _():
            _conv_slice(nidx, eidx, lo[s][0], lo[s][1], base, bl, sbuf, zbuf,
                        rbuf0, rbuf1, sem0, sem1, ef_sh, out_sh)

        @pl.when(cid == 1)
        def _():
            _conv_slice(nidx, eidx, hi[s][0], hi[s][1], base, bl, sbuf, zbuf,
                        rbuf0, rbuf1, sem0, sem1, ef_sh, out_sh)


@functools.partial(
    pl.kernel,
    mesh=_mesh,
    compiler_params=_sc_params,
    out_type=[jax.ShapeDtypeStruct((NP, W), jnp.float32)] * 4,
    scratch_types=_SC_SCRATCH,
)
def _cand_conv(c0, c1, c2, c3, nidx_h, eidx_h, binv_h,
               oc0, oc1, oc2, oc3,
               nidx, eidx, rbuf0, rbuf1, sbuf, zbuf, bl, ef_sh, out_sh,
               sem0, sem1):
    cid = lax.axis_index("c")
    sid = lax.axis_index("s")
    base = sid * RPT

    pltpu.sync_copy(nidx_h.at[sid], nidx)
    pltpu.sync_copy(eidx_h.at[sid], eidx)
    pltpu.sync_copy(binv_h.at[pl.ds(base, RPT)], bl)
    _fill_zbuf(zbuf)

    lo = [(c0, oc0), (c1, oc1)]
    hi = [(c2, oc2), (c3, oc3)]
    for s in range(2):
        @pl.when(cid == 0)
        def _():
            _conv_slice(nidx, eidx, lo[s][0], lo[s][1], base, bl, sbuf, zbuf,
                        rbuf0, rbuf1, sem0, sem1, ef_sh, out_sh)

        @pl.when(cid == 1)
        def _():
            _conv_slice(nidx, eidx, hi[s][0], hi[s][1], base, bl, sbuf, zbuf,
                        rbuf0, rbuf1, sem0, sem1, ef_sh, out_sh)


_BR = 1024


def _tc1_body(x_ref, st_ref, wt_ref, *outs):
    cat = jnp.concatenate([x_ref[...], st_ref[...]], axis=1)
    xw = jnp.dot(cat, wt_ref[...], preferred_element_type=jnp.float32)
    for k, o in enumerate(outs):
        o[...] = xw[:, k * W:(k + 1) * W]


def _layernorm(a, g, b):
    mu = jnp.mean(a, axis=-1, keepdims=True)
    var = jnp.mean((a - mu) ** 2, axis=-1, keepdims=True)
    return (a - mu) / jnp.sqrt(var + 1e-5) * g + b


def _tc2_body(x_ref, st_ref, o0, o1, o2, o3, o4, o5, o6, o7, dinv_ref,
              gb, gg, gbe, crwt_ref, rb, z_o, res_o, c0_o, c1_o, c2_o, c3_o):
    x = x_ref[...]
    st = st_ref[...]
    cat = jnp.concatenate([x, st], axis=1)
    conv = jnp.concatenate(
        [o[...] for o in (o0, o1, o2, o3, o4, o5, o6, o7)], axis=1)
    conv = conv * dinv_ref[...] + gb[...]
    g = _layernorm(jax.nn.relu(cat + conv), gg[...], gbe[...])
    zr = jax.nn.sigmoid(g)
    z = zr[:, 0:HID]
    r = zr[:, HID:]
    ci = jnp.concatenate([x, r * st], axis=1)
    both = jnp.dot(ci, crwt_ref[...], preferred_element_type=jnp.float32)
    z_o[...] = z
    res_o[...] = both[:, HID:] + rb[...]
    for k, o in enumerate((c0_o, c1_o, c2_o, c3_o)):
        o[...] = both[:, k * W:(k + 1) * W]


def _tc3_body(z_ref, res_ref, st_ref, oc0, oc1, oc2, oc3, dinv_ref,
              cb, cg, cbe, h_o):
    conv = jnp.concatenate([o[...] for o in (oc0, oc1, oc2, oc3)], axis=1)
    conv = conv * dinv_ref[...] + cb[...]
    c = _layernorm(jax.nn.relu(res_ref[...] + conv), cg[...], cbe[...])
    hc = jnp.tanh(c)
    z = z_ref[...]
    h_o[...] = (1.0 - z) * st_ref[...] + z * hc


def _row_spec(w):
    return pl.BlockSpec((_BR, w), lambda i: (i, 0))


def _full_spec(shape):
    return pl.BlockSpec(shape, lambda i: tuple(0 for _ in shape))


def kernel(x, state, hyperedge_index, hyperedge_weight, gate_W, gate_b,
           gate_g, gate_beta, cand_W, cand_b, cand_g, cand_beta, res_W,
           res_b):
    f32 = jnp.float32
    node = hyperedge_index[0]
    edge = hyperedge_index[1]
    pad = PAD_BASE + (jnp.arange(NNZ_PAD - NNZ, dtype=jnp.int32) % 16)
    node_r = jnp.concatenate([node, pad]).reshape(TPS, NCH, CHUNK)
    edge_r = jnp.concatenate([edge, pad]).reshape(TPS, NCH, CHUNK)
    hwp = jnp.concatenate(
        [hyperedge_weight, jnp.zeros((NP - hyperedge_weight.shape[0],), f32)])

    gwt = gate_W.T
    xws = pl.pallas_call(
        _tc1_body,
        grid=(NP // _BR,),
        in_specs=[_row_spec(IN_F), _row_spec(HID), _full_spec((256, 256))],
        out_specs=[_row_spec(W)] * 8,
        out_shape=[jax.ShapeDtypeStruct((NP, W), f32)] * 8,
    )(x, state, gwt)

    *os_, dinv, binv = _gate_conv(*xws, node_r, edge_r, hwp)
    dinv2 = dinv.reshape(NP, 1)

    crwt = jnp.concatenate([cand_W, res_W], axis=0).T
    z, res, c0, c1, c2, c3 = pl.pallas_call(
        _tc2_body,
        grid=(NP // _BR,),
        in_specs=[_row_spec(IN_F), _row_spec(HID)] + [_row_spec(W)] * 8 +
                 [_row_spec(1), _full_spec((1, 256)), _full_spec((1, 256)),
                  _full_spec((1, 256)), _full_spec((256, 256)),
                  _full_spec((1, HID))],
        out_specs=[_row_spec(HID), _row_spec(HID)] + [_row_spec(W)] * 4,
        out_shape=[jax.ShapeDtypeStruct((N_NODES, HID), f32),
                   jax.ShapeDtypeStruct((N_NODES, HID), f32)] +
                  [jax.ShapeDtypeStruct((NP, W), f32)] * 4,
    )(x, state, *os_, dinv2, gate_b.reshape(1, 256),
      gate_g.reshape(1, 256), gate_beta.reshape(1, 256), crwt,
      res_b.reshape(1, HID))

    ocs = _cand_conv(c0, c1, c2, c3, node_r, edge_r, binv)

    h = pl.pallas_call(
        _tc3_body,
        grid=(NP // _BR,),
        in_specs=[_row_spec(HID), _row_spec(HID), _row_spec(HID)] +
                 [_row_spec(W)] * 4 +
                 [_row_spec(1), _full_spec((1, HID)), _full_spec((1, HID)),
                  _full_spec((1, HID))],
        out_specs=_row_spec(HID),
        out_shape=jax.ShapeDtypeStruct((N_NODES, HID), f32),
    )(z, res, state, *ocs, dinv2, cand_b.reshape(1, HID),
      cand_g.reshape(1, HID), cand_beta.reshape(1, HID))
    return h


# speedup vs baseline: 10.0519x; 1.2398x over previous
import functools

import jax
import jax.numpy as jnp
from jax import lax
from jax.experimental import pallas as pl
from jax.experimental.pallas import tpu as pltpu
from jax.experimental.pallas import tpu_sc as plsc

N_NODES = 10000
NNZ = 320000
IN_F = 128
HID = 128

NP = 10240
W = 32
TPS = 16
CHUNK = 128
NNZ_PAD = 327680
NCH = NNZ_PAD // (TPS * CHUNK)
RPT = NP // TPS
PAD_BASE = 10000
K = 4
ZR = 64

_mesh = plsc.VectorSubcoreMesh(core_axis_name="c", subcore_axis_name="s")
_sc_params = pltpu.CompilerParams(use_tc_tiling_on_sc=False)


def _zero_vec(ref, n):
    z = jnp.zeros((16,), jnp.float32)

    def body(i, _):
        ref[pl.ds(i * 16, 16)] = z
        return 0

    lax.fori_loop(0, n // 16, body, 0)


def _fill_zbuf(zbuf):
    z = jnp.zeros((16,), jnp.float32)
    for r in range(ZR):
        for c in range(W // 16):
            zbuf[r, pl.ds(c * 16, 16)] = z


def _zero_shared_rows(zbuf, sh_a, sh_b, base, sem_a, sem_b):
    n = RPT // ZR

    def fire(b, _):
        pltpu.async_copy(zbuf, sh_a.at[pl.ds(base + b * ZR, ZR), :], sem_a)
        pltpu.async_copy(zbuf, sh_b.at[pl.ds(base + b * ZR, ZR), :], sem_b)
        return 0

    lax.fori_loop(0, n, fire, 0)

    def drain(b, _):
        pltpu.make_async_copy(
            zbuf, sh_a.at[pl.ds(base + b * ZR, ZR), :], sem_a).wait()
        pltpu.make_async_copy(
            zbuf, sh_b.at[pl.ds(base + b * ZR, ZR), :], sem_b).wait()
        return 0

    lax.fori_loop(0, n, drain, 0)


def _pair_sweep(nidx, eidx, src, dst, rbufs, gsems, ssems):
    for i in range(K):
        pltpu.async_copy(src.at[nidx.at[i]], rbufs[i], gsems[i])

    def body(g, _):
        j = g * K
        for i in range(K):
            pltpu.make_async_copy(
                src.at[nidx.at[j + i]], rbufs[i], gsems[i]).wait()
            pltpu.async_copy(
                rbufs[i], dst.at[eidx.at[j + i]], ssems[i], add=True)
        for i in range(K):
            nj = j + i + K

            @pl.when(nj < NCH)
            def _():
                pltpu.make_async_copy(
                    rbufs[i], dst.at[eidx.at[j + i]], ssems[i]).wait()
                pltpu.async_copy(src.at[nidx.at[nj]], rbufs[i], gsems[i])
        return 0

    lax.fori_loop(0, NCH // K, body, 0)
    for i in range(K):
        cj = NCH - K + i
        pltpu.make_async_copy(
            rbufs[i], dst.at[eidx.at[cj]], ssems[i]).wait()


def _scale_rows(ef_sh, base, bl, sbufs, gsems, ssems):
    nblk = RPT // 16

    def blk(b):
        return ef_sh.at[pl.ds(base + b * 16, 16), :]

    for i in range(K):
        pltpu.async_copy(blk(i), sbufs[i], gsems[i])

    def body(g, _):
        j = g * K
        for i in range(K):
            b = j + i
            pltpu.make_async_copy(blk(b), sbufs[i], gsems[i]).wait()
            bv16 = bl[pl.ds(b * 16, 16)]
            for r in range(16):
                bvec = lax.broadcast(bv16[r], (16,))
                for c in range(W // 16):
                    sbufs[i][r, pl.ds(c * 16, 16)] = (
                        sbufs[i][r, pl.ds(c * 16, 16)] * bvec)
            pltpu.async_copy(sbufs[i], blk(b), ssems[i])
        for i in range(K):
            nb = j + i + K

            @pl.when(nb < nblk)
            def _():
                pltpu.make_async_copy(sbufs[i], blk(j + i), ssems[i]).wait()
                pltpu.async_copy(blk(nb), sbufs[i], gsems[i])
        return 0

    lax.fori_loop(0, nblk // K, body, 0)
    for i in range(K):
        pltpu.make_async_copy(sbufs[i], blk(nblk - K + i), ssems[i]).wait()


def _conv_slice(nidx, eidx, tbl, out_hbm, base, bl, sbufs, zbuf,
                rbufs, gsems, ssems, ef_sh, out_sh):
    _zero_shared_rows(zbuf, ef_sh, out_sh, base, gsems[0], gsems[1])
    plsc.subcore_barrier()
    _pair_sweep(nidx, eidx, tbl, ef_sh, rbufs, gsems, ssems)
    plsc.subcore_barrier()
    _scale_rows(ef_sh, base, bl, sbufs, gsems, ssems)
    plsc.subcore_barrier()
    _pair_sweep(eidx, nidx, ef_sh, out_sh, rbufs, gsems, ssems)
    plsc.subcore_barrier()
    pltpu.sync_copy(out_sh.at[pl.ds(base, RPT), :],
                    out_hbm.at[pl.ds(base, RPT), :])
    plsc.subcore_barrier()


_SC_SCRATCH = (
    [pltpu.VMEM((NCH, CHUNK), jnp.int32)] * 2
    + [pltpu.VMEM((CHUNK, W), jnp.float32)] * K
    + [pltpu.VMEM((16, W), jnp.float32)] * K
    + [pltpu.VMEM((ZR, W), jnp.float32)]
    + [pltpu.VMEM((RPT,), jnp.float32)]
    + [pltpu.VMEM_SHARED((NP, W), jnp.float32)] * 2
    + [pltpu.SemaphoreType.DMA] * (2 * K)
)


@functools.partial(
    pl.kernel,
    mesh=_mesh,
    compiler_params=_sc_params,
    out_type=(
        [jax.ShapeDtypeStruct((NP, W), jnp.float32)] * 8
        + [jax.ShapeDtypeStruct((NP,), jnp.float32)] * 2
    ),
    scratch_types=_SC_SCRATCH + (
        [pltpu.VMEM((CHUNK,), jnp.float32)] * K
        + [pltpu.VMEM((CHUNK,), jnp.float32)]
        + [pltpu.VMEM((RPT,), jnp.float32)]
        + [pltpu.VMEM_SHARED((NP,), jnp.float32)] * 2
        + [pltpu.SemaphoreType.DMA] * K
    ),
)
def _gate_conv(xw0, xw1, xw2, xw3, xw4, xw5, xw6, xw7, nidx_h, eidx_h, hw_h,
               o0, o1, o2, o3, o4, o5, o6, o7, dinv_h, binv_h,
               nidx, eidx, rb0, rb1, rb2, rb3, sb0, sb1, sb2, sb3, zbuf, bl,
               ef_sh, out_sh, g0, g1, g2, g3, s0, s1, s2, s3,
               hb0, hb1, hb2, hb3, ones, dl, d_sh, bd_sh, b0, b1, b2, b3):
    cid = lax.axis_index("c")
    sid = lax.axis_index("s")
    base = sid * RPT
    rbufs = [rb0, rb1, rb2, rb3]
    sbufs = [sb0, sb1, sb2, sb3]
    gsems = [g0, g1, g2, g3]
    ssems = [s0, s1, s2, s3]
    bsems = [b0, b1, b2, b3]
    hwbufs = [hb0, hb1, hb2, hb3]

    pltpu.sync_copy(nidx_h.at[sid], nidx)
    pltpu.sync_copy(eidx_h.at[sid], eidx)

    _fill_zbuf(zbuf)
    one = jnp.ones((16,), jnp.float32)
    for i in range(CHUNK // 16):
        ones[pl.ds(i * 16, 16)] = one

    _zero_vec(bl, RPT)
    pltpu.sync_copy(bl, d_sh.at[pl.ds(base, RPT)])
    pltpu.sync_copy(bl, bd_sh.at[pl.ds(base, RPT)])
    plsc.subcore_barrier()

    for i in range(K):
        pltpu.async_copy(hw_h.at[eidx.at[i]], hwbufs[i], gsems[i])

    def hist(g, _):
        j = g * K
        for i in range(K):
            pltpu.make_async_copy(
                hw_h.at[eidx.at[j + i]], hwbufs[i], gsems[i]).wait()
            pltpu.async_copy(
                hwbufs[i], d_sh.at[nidx.at[j + i]], ssems[i], add=True)
            pltpu.async_copy(
                ones, bd_sh.at[eidx.at[j + i]], bsems[i], add=True)
        for i in range(K):
            nj = j + i + K

            @pl.when(nj < NCH)
            def _():
                pltpu.make_async_copy(
                    hwbufs[i], d_sh.at[nidx.at[j + i]], ssems[i]).wait()
                pltpu.make_async_copy(
                    ones, bd_sh.at[eidx.at[j + i]], bsems[i]).wait()
                pltpu.async_copy(hw_h.at[eidx.at[nj]], hwbufs[i], gsems[i])
        return 0

    lax.fori_loop(0, NCH // K, hist, 0)
    for i in range(K):
        cj = NCH - K + i
        pltpu.make_async_copy(
            hwbufs[i], d_sh.at[nidx.at[cj]], ssems[i]).wait()
        pltpu.make_async_copy(ones, bd_sh.at[eidx.at[cj]], bsems[i]).wait()
    plsc.subcore_barrier()

    pltpu.sync_copy(bd_sh.at[pl.ds(base, RPT)], bl)
    pltpu.sync_copy(d_sh.at[pl.ds(base, RPT)], dl)

    def recip(i, _):
        bd = bl[pl.ds(i * 16, 16)]
        bl[pl.ds(i * 16, 16)] = jnp.where(bd > 0.0, 1.0 / bd, 0.0)
        dv = dl[pl.ds(i * 16, 16)]
        dl[pl.ds(i * 16, 16)] = jnp.where(dv > 0.0, 1.0 / dv, 0.0)
        return 0

    lax.fori_loop(0, RPT // 16, recip, 0)

    @pl.when(cid == 0)
    def _():
        pltpu.sync_copy(bl, binv_h.at[pl.ds(base, RPT)])
        pltpu.sync_copy(dl, dinv_h.at[pl.ds(base, RPT)])

    lo = [(xw0, o0), (xw1, o1), (xw2, o2), (xw3, o3)]
    hi = [(xw4, o4), (xw5, o5), (xw6, o6), (xw7, o7)]
    for s in range(4):
        @pl.when(cid == 0)
        def _():
            _conv_slice(nidx, eidx, lo[s][0], lo[s][1], base, bl, sbufs,
                        zbuf, rbufs, gsems, ssems, ef_sh, out_sh)

        @pl.when(cid == 1)
        def _():
            _conv_slice(nidx, eidx, hi[s][0], hi[s][1], base, bl, sbufs,
                        zbuf, rbufs, gsems, ssems, ef_sh, out_sh)


@functools.partial(
    pl.kernel,
    mesh=_mesh,
    compiler_params=_sc_params,
    out_type=[jax.ShapeDtypeStruct((NP, W), jnp.float32)] * 4,
    scratch_types=_SC_SCRATCH,
)
def _cand_conv(c0, c1, c2, c3, nidx_h, eidx_h, binv_h,
               oc0, oc1, oc2, oc3,
               nidx, eidx, rb0, rb1, rb2, rb3, sb0, sb1, sb2, sb3, zbuf, bl,
               ef_sh, out_sh, g0, g1, g2, g3, s0, s1, s2, s3):
    cid = lax.axis_index("c")
    sid = lax.axis_index("s")
    base = sid * RPT
    rbufs = [rb0, rb1, rb2, rb3]
    sbufs = [sb0, sb1, sb2, sb3]
    gsems = [g0, g1, g2, g3]
    ssems = [s0, s1, s2, s3]

    pltpu.sync_copy(nidx_h.at[sid], nidx)
    pltpu.sync_copy(eidx_h.at[sid], eidx)
    pltpu.sync_copy(binv_h.at[pl.ds(base, RPT)], bl)
    _fill_zbuf(zbuf)

    lo = [(c0, oc0), (c1, oc1)]
    hi = [(c2, oc2), (c3, oc3)]
    for s in range(2):
        @pl.when(cid == 0)
        def _():
            _conv_slice(nidx, eidx, lo[s][0], lo[s][1], base, bl, sbufs,
                        zbuf, rbufs, gsems, ssems, ef_sh, out_sh)

        @pl.when(cid == 1)
        def _():
            _conv_slice(nidx, eidx, hi[s][0], hi[s][1], base, bl, sbufs,
                        zbuf, rbufs, gsems, ssems, ef_sh, out_sh)


_BR = 1024


def _tc1_body(x_ref, st_ref, wt_ref, *outs):
    cat = jnp.concatenate([x_ref[...], st_ref[...]], axis=1)
    xw = jnp.dot(cat, wt_ref[...], preferred_element_type=jnp.float32)
    for k, o in enumerate(outs):
        o[...] = xw[:, k * W:(k + 1) * W]


def _layernorm(a, g, b):
    mu = jnp.mean(a, axis=-1, keepdims=True)
    var = jnp.mean((a - mu) ** 2, axis=-1, keepdims=True)
    return (a - mu) / jnp.sqrt(var + 1e-5) * g + b


def _tc2_body(x_ref, st_ref, o0, o1, o2, o3, o4, o5, o6, o7, dinv_ref,
              gb, gg, gbe, crwt_ref, rb, z_o, res_o, c0_o, c1_o, c2_o, c3_o):
    x = x_ref[...]
    st = st_ref[...]
    cat = jnp.concatenate([x, st], axis=1)
    conv = jnp.concatenate(
        [o[...] for o in (o0, o1, o2, o3, o4, o5, o6, o7)], axis=1)
    conv = conv * dinv_ref[...] + gb[...]
    g = _layernorm(jax.nn.relu(cat + conv), gg[...], gbe[...])
    zr = jax.nn.sigmoid(g)
    z = zr[:, 0:HID]
    r = zr[:, HID:]
    ci = jnp.concatenate([x, r * st], axis=1)
    both = jnp.dot(ci, crwt_ref[...], preferred_element_type=jnp.float32)
    z_o[...] = z
    res_o[...] = both[:, HID:] + rb[...]
    for k, o in enumerate((c0_o, c1_o, c2_o, c3_o)):
        o[...] = both[:, k * W:(k + 1) * W]


def _tc3_body(z_ref, res_ref, st_ref, oc0, oc1, oc2, oc3, dinv_ref,
              cb, cg, cbe, h_o):
    conv = jnp.concatenate([o[...] for o in (oc0, oc1, oc2, oc3)], axis=1)
    conv = conv * dinv_ref[...] + cb[...]
    c = _layernorm(jax.nn.relu(res_ref[...] + conv), cg[...], cbe[...])
    hc = jnp.tanh(c)
    z = z_ref[...]
    h_o[...] = (1.0 - z) * st_ref[...] + z * hc


def _row_spec(w):
    return pl.BlockSpec((_BR, w), lambda i: (i, 0))


def _full_spec(shape):
    return pl.BlockSpec(shape, lambda i: tuple(0 for _ in shape))


def kernel(x, state, hyperedge_index, hyperedge_weight, gate_W, gate_b,
           gate_g, gate_beta, cand_W, cand_b, cand_g, cand_beta, res_W,
           res_b):
    f32 = jnp.float32
    node = hyperedge_index[0]
    edge = hyperedge_index[1]
    pad = PAD_BASE + (jnp.arange(NNZ_PAD - NNZ, dtype=jnp.int32) % 16)
    node_r = jnp.concatenate([node, pad]).reshape(TPS, NCH, CHUNK)
    edge_r = jnp.concatenate([edge, pad]).reshape(TPS, NCH, CHUNK)
    hwp = jnp.concatenate(
        [hyperedge_weight, jnp.zeros((NP - hyperedge_weight.shape[0],), f32)])

    gwt = gate_W.T
    xws = pl.pallas_call(
        _tc1_body,
        grid=(NP // _BR,),
        in_specs=[_row_spec(IN_F), _row_spec(HID), _full_spec((256, 256))],
        out_specs=[_row_spec(W)] * 8,
        out_shape=[jax.ShapeDtypeStruct((NP, W), f32)] * 8,
    )(x, state, gwt)

    *os_, dinv, binv = _gate_conv(*xws, node_r, edge_r, hwp)
    dinv2 = dinv.reshape(NP, 1)

    crwt = jnp.concatenate([cand_W, res_W], axis=0).T
    z, res, c0, c1, c2, c3 = pl.pallas_call(
        _tc2_body,
        grid=(NP // _BR,),
        in_specs=[_row_spec(IN_F), _row_spec(HID)] + [_row_spec(W)] * 8 +
                 [_row_spec(1), _full_spec((1, 256)), _full_spec((1, 256)),
                  _full_spec((1, 256)), _full_spec((256, 256)),
                  _full_spec((1, HID))],
        out_specs=[_row_spec(HID), _row_spec(HID)] + [_row_spec(W)] * 4,
        out_shape=[jax.ShapeDtypeStruct((N_NODES, HID), f32),
                   jax.ShapeDtypeStruct((N_NODES, HID), f32)] +
                  [jax.ShapeDtypeStruct((NP, W), f32)] * 4,
    )(x, state, *os_, dinv2, gate_b.reshape(1, 256),
      gate_g.reshape(1, 256), gate_beta.reshape(1, 256), crwt,
      res_b.reshape(1, HID))

    ocs = _cand_conv(c0, c1, c2, c3, node_r, edge_r, binv)

    h = pl.pallas_call(
        _tc3_body,
        grid=(NP // _BR,),
        in_specs=[_row_spec(HID), _row_spec(HID), _row_spec(HID)] +
                 [_row_spec(W)] * 4 +
                 [_row_spec(1), _full_spec((1, HID)), _full_spec((1, HID)),
                  _full_spec((1, HID))],
        out_specs=_row_spec(HID),
        out_shape=jax.ShapeDtypeStruct((N_NODES, HID), f32),
    )(z, res, state, *ocs, dinv2, cand_b.reshape(1, HID),
      cand_g.reshape(1, HID), cand_beta.reshape(1, HID))
    return h


# speedup vs baseline: 11.8758x; 1.1814x over previous
import functools

import jax
import jax.numpy as jnp
from jax import lax
from jax.experimental import pallas as pl
from jax.experimental.pallas import tpu as pltpu
from jax.experimental.pallas import tpu_sc as plsc

N_NODES = 10000
NNZ = 320000
IN_F = 128
HID = 128

NP = 10240
W = 32
TPS = 16
CHUNK = 128
NNZ_PAD = 327680
NCH = NNZ_PAD // (TPS * CHUNK)
RPT = NP // TPS
PAD_BASE = 10000
K = 4
ZR = 64

_mesh = plsc.VectorSubcoreMesh(core_axis_name="c", subcore_axis_name="s")
_sc_params = pltpu.CompilerParams(use_tc_tiling_on_sc=False)


def _zero_vec(ref, n):
    z = jnp.zeros((16,), jnp.float32)

    def body(i, _):
        ref[pl.ds(i * 16, 16)] = z
        return 0

    lax.fori_loop(0, n // 16, body, 0)


def _fill_zbuf(zbuf):
    z = jnp.zeros((16,), jnp.float32)
    for r in range(ZR):
        for c in range(W // 16):
            zbuf[r, pl.ds(c * 16, 16)] = z


def _zero_shared_rows(zbuf, sh_a, sh_b, base, sem_a, sem_b):
    n = RPT // ZR

    def fire(b, _):
        pltpu.async_copy(zbuf, sh_a.at[pl.ds(base + b * ZR, ZR), :], sem_a)
        pltpu.async_copy(zbuf, sh_b.at[pl.ds(base + b * ZR, ZR), :], sem_b)
        return 0

    lax.fori_loop(0, n, fire, 0)

    def drain(b, _):
        pltpu.make_async_copy(
            zbuf, sh_a.at[pl.ds(base + b * ZR, ZR), :], sem_a).wait()
        pltpu.make_async_copy(
            zbuf, sh_b.at[pl.ds(base + b * ZR, ZR), :], sem_b).wait()
        return 0

    lax.fori_loop(0, n, drain, 0)


def _pair_sweep(nidx, eidx, src, dst, rbufs, gsems, ssems, hist=None):
    if hist is not None:
        hw_h, hwbufs, ones, d_sh, bd_sh, bsems, csems = hist

    for i in range(K):
        pltpu.async_copy(src.at[nidx.at[i]], rbufs[i], gsems[i])
        if hist is not None:
            pltpu.async_copy(hw_h.at[eidx.at[i]], hwbufs[i], csems[i])

    def body(g, _):
        j = g * K
        for i in range(K):
            pltpu.make_async_copy(
                src.at[nidx.at[j + i]], rbufs[i], gsems[i]).wait()
            pltpu.async_copy(
                rbufs[i], dst.at[eidx.at[j + i]], ssems[i], add=True)
            if hist is not None:
                pltpu.make_async_copy(
                    hw_h.at[eidx.at[j + i]], hwbufs[i], csems[i]).wait()
                pltpu.async_copy(
                    hwbufs[i], d_sh.at[nidx.at[j + i]], csems[i], add=True)
                pltpu.async_copy(
                    ones, bd_sh.at[eidx.at[j + i]], bsems[i], add=True)
        for i in range(K):
            nj = j + i + K

            @pl.when(nj < NCH)
            def _():
                pltpu.make_async_copy(
                    rbufs[i], dst.at[eidx.at[j + i]], ssems[i]).wait()
                pltpu.async_copy(src.at[nidx.at[nj]], rbufs[i], gsems[i])
                if hist is not None:
                    pltpu.make_async_copy(
                        hwbufs[i], d_sh.at[nidx.at[j + i]], csems[i]).wait()
                    pltpu.make_async_copy(
                        ones, bd_sh.at[eidx.at[j + i]], bsems[i]).wait()
                    pltpu.async_copy(
                        hw_h.at[eidx.at[nj]], hwbufs[i], csems[i])
        return 0

    lax.fori_loop(0, NCH // K, body, 0)
    for i in range(K):
        cj = NCH - K + i
        pltpu.make_async_copy(
            rbufs[i], dst.at[eidx.at[cj]], ssems[i]).wait()
        if hist is not None:
            pltpu.make_async_copy(
                hwbufs[i], d_sh.at[nidx.at[cj]], csems[i]).wait()
            pltpu.make_async_copy(
                ones, bd_sh.at[eidx.at[cj]], bsems[i]).wait()


def _scale_rows(ef_sh, base, bl, sbufs, gsems, ssems):
    nblk = RPT // 16

    def blk(b):
        return ef_sh.at[pl.ds(base + b * 16, 16), :]

    for i in range(K):
        pltpu.async_copy(blk(i), sbufs[i], gsems[i])

    def body(g, _):
        j = g * K
        for i in range(K):
            b = j + i
            pltpu.make_async_copy(blk(b), sbufs[i], gsems[i]).wait()
            bv16 = bl[pl.ds(b * 16, 16)]
            for r in range(16):
                bvec = lax.broadcast(bv16[r], (16,))
                for c in range(W // 16):
                    sbufs[i][r, pl.ds(c * 16, 16)] = (
                        sbufs[i][r, pl.ds(c * 16, 16)] * bvec)
            pltpu.async_copy(sbufs[i], blk(b), ssems[i])
        for i in range(K):
            nb = j + i + K

            @pl.when(nb < nblk)
            def _():
                pltpu.make_async_copy(sbufs[i], blk(j + i), ssems[i]).wait()
                pltpu.async_copy(blk(nb), sbufs[i], gsems[i])
        return 0

    lax.fori_loop(0, nblk // K, body, 0)
    for i in range(K):
        pltpu.make_async_copy(sbufs[i], blk(nblk - K + i), ssems[i]).wait()


def _conv_slice(nidx, eidx, tbl, out_hbm, base, bl, sbufs, zbuf,
                rbufs, gsems, ssems, ef_sh, out_sh,
                hist=None, after_stage1=None):
    _zero_shared_rows(zbuf, ef_sh, out_sh, base, gsems[0], gsems[1])
    plsc.subcore_barrier()
    _pair_sweep(nidx, eidx, tbl, ef_sh, rbufs, gsems, ssems, hist=hist)
    plsc.subcore_barrier()
    if after_stage1 is not None:
        after_stage1()
    _scale_rows(ef_sh, base, bl, sbufs, gsems, ssems)
    plsc.subcore_barrier()
    _pair_sweep(eidx, nidx, ef_sh, out_sh, rbufs, gsems, ssems)
    plsc.subcore_barrier()
    pltpu.sync_copy(out_sh.at[pl.ds(base, RPT), :],
                    out_hbm.at[pl.ds(base, RPT), :])
    plsc.subcore_barrier()


_SC_SCRATCH = (
    [pltpu.VMEM((NCH, CHUNK), jnp.int32)] * 2
    + [pltpu.VMEM((CHUNK, W), jnp.float32)] * K
    + [pltpu.VMEM((16, W), jnp.float32)] * K
    + [pltpu.VMEM((ZR, W), jnp.float32)]
    + [pltpu.VMEM((RPT,), jnp.float32)]
    + [pltpu.VMEM_SHARED((NP, W), jnp.float32)] * 2
    + [pltpu.SemaphoreType.DMA] * (2 * K)
)


@functools.partial(
    pl.kernel,
    mesh=_mesh,
    compiler_params=_sc_params,
    out_type=(
        [jax.ShapeDtypeStruct((NP, W), jnp.float32)] * 8
        + [jax.ShapeDtypeStruct((NP,), jnp.float32)] * 2
    ),
    scratch_types=_SC_SCRATCH + (
        [pltpu.VMEM((CHUNK,), jnp.float32)] * K
        + [pltpu.VMEM((CHUNK,), jnp.float32)]
        + [pltpu.VMEM((RPT,), jnp.float32)]
        + [pltpu.VMEM_SHARED((NP,), jnp.float32)] * 2
        + [pltpu.SemaphoreType.DMA] * (2 * K)
    ),
)
def _gate_conv(xw0, xw1, xw2, xw3, xw4, xw5, xw6, xw7, nidx_h, eidx_h, hw_h,
               o0, o1, o2, o3, o4, o5, o6, o7, dinv_h, binv_h,
               nidx, eidx, rb0, rb1, rb2, rb3, sb0, sb1, sb2, sb3, zbuf, bl,
               ef_sh, out_sh, g0, g1, g2, g3, s0, s1, s2, s3,
               hb0, hb1, hb2, hb3, ones, dl, d_sh, bd_sh,
               b0, b1, b2, b3, q0, q1, q2, q3):
    cid = lax.axis_index("c")
    sid = lax.axis_index("s")
    base = sid * RPT
    rbufs = [rb0, rb1, rb2, rb3]
    sbufs = [sb0, sb1, sb2, sb3]
    gsems = [g0, g1, g2, g3]
    ssems = [s0, s1, s2, s3]
    bsems = [b0, b1, b2, b3]
    csems = [q0, q1, q2, q3]
    hwbufs = [hb0, hb1, hb2, hb3]

    pltpu.sync_copy(nidx_h.at[sid], nidx)
    pltpu.sync_copy(eidx_h.at[sid], eidx)

    _fill_zbuf(zbuf)
    one = jnp.ones((16,), jnp.float32)
    for i in range(CHUNK // 16):
        ones[pl.ds(i * 16, 16)] = one

    _zero_vec(bl, RPT)
    pltpu.sync_copy(bl, d_sh.at[pl.ds(base, RPT)])
    pltpu.sync_copy(bl, bd_sh.at[pl.ds(base, RPT)])

    hist_args = (hw_h, hwbufs, ones, d_sh, bd_sh, bsems, csems)

    def finish_hist():
        pltpu.sync_copy(bd_sh.at[pl.ds(base, RPT)], bl)
        pltpu.sync_copy(d_sh.at[pl.ds(base, RPT)], dl)

        def recip(i, _):
            bd = bl[pl.ds(i * 16, 16)]
            bl[pl.ds(i * 16, 16)] = jnp.where(bd > 0.0, 1.0 / bd, 0.0)
            dv = dl[pl.ds(i * 16, 16)]
            dl[pl.ds(i * 16, 16)] = jnp.where(dv > 0.0, 1.0 / dv, 0.0)
            return 0

        lax.fori_loop(0, RPT // 16, recip, 0)

        @pl.when(cid == 0)
        def _():
            pltpu.sync_copy(bl, binv_h.at[pl.ds(base, RPT)])
            pltpu.sync_copy(dl, dinv_h.at[pl.ds(base, RPT)])

    lo = [(xw0, o0), (xw1, o1), (xw2, o2), (xw3, o3)]
    hi = [(xw4, o4), (xw5, o5), (xw6, o6), (xw7, o7)]
    for s in range(4):
        h = hist_args if s == 0 else None
        fh = finish_hist if s == 0 else None

        @pl.when(cid == 0)
        def _():
            _conv_slice(nidx, eidx, lo[s][0], lo[s][1], base, bl, sbufs,
                        zbuf, rbufs, gsems, ssems, ef_sh, out_sh,
                        hist=h, after_stage1=fh)

        @pl.when(cid == 1)
        def _():
            _conv_slice(nidx, eidx, hi[s][0], hi[s][1], base, bl, sbufs,
                        zbuf, rbufs, gsems, ssems, ef_sh, out_sh,
                        hist=h, after_stage1=fh)


@functools.partial(
    pl.kernel,
    mesh=_mesh,
    compiler_params=_sc_params,
    out_type=[jax.ShapeDtypeStruct((NP, W), jnp.float32)] * 4,
    scratch_types=_SC_SCRATCH,
)
def _cand_conv(c0, c1, c2, c3, nidx_h, eidx_h, binv_h,
               oc0, oc1, oc2, oc3,
               nidx, eidx, rb0, rb1, rb2, rb3, sb0, sb1, sb2, sb3, zbuf, bl,
               ef_sh, out_sh, g0, g1, g2, g3, s0, s1, s2, s3):
    cid = lax.axis_index("c")
    sid = lax.axis_index("s")
    base = sid * RPT
    rbufs = [rb0, rb1, rb2, rb3]
    sbufs = [sb0, sb1, sb2, sb3]
    gsems = [g0, g1, g2, g3]
    ssems = [s0, s1, s2, s3]

    pltpu.sync_copy(nidx_h.at[sid], nidx)
    pltpu.sync_copy(eidx_h.at[sid], eidx)
    pltpu.sync_copy(binv_h.at[pl.ds(base, RPT)], bl)
    _fill_zbuf(zbuf)

    lo = [(c0, oc0), (c1, oc1)]
    hi = [(c2, oc2), (c3, oc3)]
    for s in range(2):
        @pl.when(cid == 0)
        def _():
            _conv_slice(nidx, eidx, lo[s][0], lo[s][1], base, bl, sbufs,
                        zbuf, rbufs, gsems, ssems, ef_sh, out_sh)

        @pl.when(cid == 1)
        def _():
            _conv_slice(nidx, eidx, hi[s][0], hi[s][1], base, bl, sbufs,
                        zbuf, rbufs, gsems, ssems, ef_sh, out_sh)


_BR = 1024


def _tc1_body(x_ref, st_ref, wt_ref, *outs):
    cat = jnp.concatenate([x_ref[...], st_ref[...]], axis=1)
    xw = jnp.dot(cat, wt_ref[...], preferred_element_type=jnp.float32)
    for k, o in enumerate(outs):
        o[...] = xw[:, k * W:(k + 1) * W]


def _layernorm(a, g, b):
    mu = jnp.mean(a, axis=-1, keepdims=True)
    var = jnp.mean((a - mu) ** 2, axis=-1, keepdims=True)
    return (a - mu) / jnp.sqrt(var + 1e-5) * g + b


def _tc2_body(x_ref, st_ref, o0, o1, o2, o3, o4, o5, o6, o7, dinv_ref,
              gb, gg, gbe, crwt_ref, rb, z_o, res_o, c0_o, c1_o, c2_o, c3_o):
    x = x_ref[...]
    st = st_ref[...]
    cat = jnp.concatenate([x, st], axis=1)
    conv = jnp.concatenate(
        [o[...] for o in (o0, o1, o2, o3, o4, o5, o6, o7)], axis=1)
    conv = conv * dinv_ref[...] + gb[...]
    g = _layernorm(jax.nn.relu(cat + conv), gg[...], gbe[...])
    zr = jax.nn.sigmoid(g)
    z = zr[:, 0:HID]
    r = zr[:, HID:]
    ci = jnp.concatenate([x, r * st], axis=1)
    both = jnp.dot(ci, crwt_ref[...], preferred_element_type=jnp.float32)
    z_o[...] = z
    res_o[...] = both[:, HID:] + rb[...]
    for k, o in enumerate((c0_o, c1_o, c2_o, c3_o)):
        o[...] = both[:, k * W:(k + 1) * W]


def _tc3_body(z_ref, res_ref, st_ref, oc0, oc1, oc2, oc3, dinv_ref,
              cb, cg, cbe, h_o):
    conv = jnp.concatenate([o[...] for o in (oc0, oc1, oc2, oc3)], axis=1)
    conv = conv * dinv_ref[...] + cb[...]
    c = _layernorm(jax.nn.relu(res_ref[...] + conv), cg[...], cbe[...])
    hc = jnp.tanh(c)
    z = z_ref[...]
    h_o[...] = (1.0 - z) * st_ref[...] + z * hc


def _row_spec(w):
    return pl.BlockSpec((_BR, w), lambda i: (i, 0))


def _full_spec(shape):
    return pl.BlockSpec(shape, lambda i: tuple(0 for _ in shape))


def kernel(x, state, hyperedge_index, hyperedge_weight, gate_W, gate_b,
           gate_g, gate_beta, cand_W, cand_b, cand_g, cand_beta, res_W,
           res_b):
    f32 = jnp.float32
    node = hyperedge_index[0]
    edge = hyperedge_index[1]
    pad = PAD_BASE + (jnp.arange(NNZ_PAD - NNZ, dtype=jnp.int32) % (NP - PAD_BASE))
    node_r = jnp.concatenate([node, pad]).reshape(TPS, NCH, CHUNK)
    edge_r = jnp.concatenate([edge, pad]).reshape(TPS, NCH, CHUNK)
    hwp = jnp.concatenate(
        [hyperedge_weight, jnp.zeros((NP - hyperedge_weight.shape[0],), f32)])

    gwt = gate_W.T
    xws = pl.pallas_call(
        _tc1_body,
        grid=(NP // _BR,),
        in_specs=[_row_spec(IN_F), _row_spec(HID), _full_spec((256, 256))],
        out_specs=[_row_spec(W)] * 8,
        out_shape=[jax.ShapeDtypeStruct((NP, W), f32)] * 8,
    )(x, state, gwt)

    *os_, dinv, binv = _gate_conv(*xws, node_r, edge_r, hwp)
    dinv2 = dinv.reshape(NP, 1)

    crwt = jnp.concatenate([cand_W, res_W], axis=0).T
    z, res, c0, c1, c2, c3 = pl.pallas_call(
        _tc2_body,
        grid=(NP // _BR,),
        in_specs=[_row_spec(IN_F), _row_spec(HID)] + [_row_spec(W)] * 8 +
                 [_row_spec(1), _full_spec((1, 256)), _full_spec((1, 256)),
                  _full_spec((1, 256)), _full_spec((256, 256)),
                  _full_spec((1, HID))],
        out_specs=[_row_spec(HID), _row_spec(HID)] + [_row_spec(W)] * 4,
        out_shape=[jax.ShapeDtypeStruct((N_NODES, HID), f32),
                   jax.ShapeDtypeStruct((N_NODES, HID), f32)] +
                  [jax.ShapeDtypeStruct((NP, W), f32)] * 4,
    )(x, state, *os_, dinv2, gate_b.reshape(1, 256),
      gate_g.reshape(1, 256), gate_beta.reshape(1, 256), crwt,
      res_b.reshape(1, HID))

    ocs = _cand_conv(c0, c1, c2, c3, node_r, edge_r, binv)

    h = pl.pallas_call(
        _tc3_body,
        grid=(NP // _BR,),
        in_specs=[_row_spec(HID), _row_spec(HID), _row_spec(HID)] +
                 [_row_spec(W)] * 4 +
                 [_row_spec(1), _full_spec((1, HID)), _full_spec((1, HID)),
                  _full_spec((1, HID))],
        out_specs=_row_spec(HID),
        out_shape=jax.ShapeDtypeStruct((N_NODES, HID), f32),
    )(z, res, state, *ocs, dinv2, cand_b.reshape(1, HID),
      cand_g.reshape(1, HID), cand_beta.reshape(1, HID))
    return h


# speedup vs baseline: 11.8773x; 1.0001x over previous
import functools

import jax
import jax.numpy as jnp
from jax import lax
from jax.experimental import pallas as pl
from jax.experimental.pallas import tpu as pltpu
from jax.experimental.pallas import tpu_sc as plsc

N_NODES = 10000
NNZ = 320000
IN_F = 128
HID = 128

NP = 10240
W = 32
TPS = 16
CHUNK = 128
NNZ_PAD = 327680
NCH = NNZ_PAD // (TPS * CHUNK)
RPT = NP // TPS
PAD_BASE = 10000
K = 4
ZR = 64

_mesh = plsc.VectorSubcoreMesh(core_axis_name="c", subcore_axis_name="s")
_sc_params = pltpu.CompilerParams(use_tc_tiling_on_sc=False)


def _zero_vec(ref, n):
    z = jnp.zeros((16,), jnp.float32)

    def body(i, _):
        ref[pl.ds(i * 16, 16)] = z
        return 0

    lax.fori_loop(0, n // 16, body, 0)


def _fill_zbuf(zbuf):
    z = jnp.zeros((16,), jnp.float32)
    for r in range(ZR):
        for c in range(W // 16):
            zbuf[r, pl.ds(c * 16, 16)] = z


def _zero_shared_rows(zbuf, sh_a, sh_b, base, sem_a, sem_b):
    n = RPT // ZR

    def fire(b, _):
        pltpu.async_copy(zbuf, sh_a.at[pl.ds(base + b * ZR, ZR), :], sem_a)
        pltpu.async_copy(zbuf, sh_b.at[pl.ds(base + b * ZR, ZR), :], sem_b)
        return 0

    lax.fori_loop(0, n, fire, 0)

    def drain(b, _):
        pltpu.make_async_copy(
            zbuf, sh_a.at[pl.ds(base + b * ZR, ZR), :], sem_a).wait()
        pltpu.make_async_copy(
            zbuf, sh_b.at[pl.ds(base + b * ZR, ZR), :], sem_b).wait()
        return 0

    lax.fori_loop(0, n, drain, 0)


def _pair_sweep(nidx, eidx, src, dst, rbufs, gsems, ssems, hist=None):
    if hist is not None:
        hw_h, hwbufs, ones, d_sh, bd_sh, bsems, csems = hist

    for i in range(K):
        pltpu.async_copy(src.at[nidx.at[i]], rbufs[i], gsems[i])
        if hist is not None:
            pltpu.async_copy(hw_h.at[eidx.at[i]], hwbufs[i], csems[i])

    def body(g, _):
        j = g * K
        for i in range(K):
            pltpu.make_async_copy(
                src.at[nidx.at[j + i]], rbufs[i], gsems[i]).wait()
            pltpu.async_copy(
                rbufs[i], dst.at[eidx.at[j + i]], ssems[i], add=True)
            if hist is not None:
                pltpu.make_async_copy(
                    hw_h.at[eidx.at[j + i]], hwbufs[i], csems[i]).wait()
                pltpu.async_copy(
                    hwbufs[i], d_sh.at[nidx.at[j + i]], csems[i], add=True)
                pltpu.async_copy(
                    ones, bd_sh.at[eidx.at[j + i]], bsems[i], add=True)
        for i in range(K):
            nj = j + i + K

            @pl.when(nj < NCH)
            def _():
                pltpu.make_async_copy(
                    rbufs[i], dst.at[eidx.at[j + i]], ssems[i]).wait()
                pltpu.async_copy(src.at[nidx.at[nj]], rbufs[i], gsems[i])
                if hist is not None:
                    pltpu.make_async_copy(
                        hwbufs[i], d_sh.at[nidx.at[j + i]], csems[i]).wait()
                    pltpu.make_async_copy(
                        ones, bd_sh.at[eidx.at[j + i]], bsems[i]).wait()
                    pltpu.async_copy(
                        hw_h.at[eidx.at[nj]], hwbufs[i], csems[i])
        return 0

    lax.fori_loop(0, NCH // K, body, 0)
    for i in range(K):
        cj = NCH - K + i
        pltpu.make_async_copy(
            rbufs[i], dst.at[eidx.at[cj]], ssems[i]).wait()
        if hist is not None:
            pltpu.make_async_copy(
                hwbufs[i], d_sh.at[nidx.at[cj]], csems[i]).wait()
            pltpu.make_async_copy(
                ones, bd_sh.at[eidx.at[cj]], bsems[i]).wait()


def _scale_rows(ef_sh, base, bl, sbufs, gsems, ssems):
    nblk = RPT // 16

    def blk(b):
        return ef_sh.at[pl.ds(base + b * 16, 16), :]

    for i in range(K):
        pltpu.async_copy(blk(i), sbufs[i], gsems[i])

    def body(g, _):
        j = g * K
        for i in range(K):
            b = j + i
            pltpu.make_async_copy(blk(b), sbufs[i], gsems[i]).wait()
            bv16 = bl[pl.ds(b * 16, 16)]
            for r in range(16):
                bvec = lax.broadcast(bv16[r], (16,))
                for c in range(W // 16):
                    sbufs[i][r, pl.ds(c * 16, 16)] = (
                        sbufs[i][r, pl.ds(c * 16, 16)] * bvec)
            pltpu.async_copy(sbufs[i], blk(b), ssems[i])
        for i in range(K):
            nb = j + i + K

            @pl.when(nb < nblk)
            def _():
                pltpu.make_async_copy(sbufs[i], blk(j + i), ssems[i]).wait()
                pltpu.async_copy(blk(nb), sbufs[i], gsems[i])
        return 0

    lax.fori_loop(0, nblk // K, body, 0)
    for i in range(K):
        pltpu.make_async_copy(sbufs[i], blk(nblk - K + i), ssems[i]).wait()


def _conv_slice(nidx, eidx, tbl, out_hbm, base, bl, sbufs, zbuf,
                rbufs, gsems, ssems, ef_sh, out_sh,
                hist=None, after_stage1=None):
    _zero_shared_rows(zbuf, ef_sh, out_sh, base, gsems[0], gsems[1])
    plsc.subcore_barrier()
    _pair_sweep(nidx, eidx, tbl, ef_sh, rbufs, gsems, ssems, hist=hist)
    plsc.subcore_barrier()
    if after_stage1 is not None:
        after_stage1()
    _scale_rows(ef_sh, base, bl, sbufs, gsems, ssems)
    plsc.subcore_barrier()
    _pair_sweep(eidx, nidx, ef_sh, out_sh, rbufs, gsems, ssems)
    plsc.subcore_barrier()
    pltpu.sync_copy(out_sh.at[pl.ds(base, RPT), :],
                    out_hbm.at[pl.ds(base, RPT), :])
    plsc.subcore_barrier()


_SC_SCRATCH = (
    [pltpu.VMEM((NCH, CHUNK), jnp.int32)] * 2
    + [pltpu.VMEM((CHUNK, W), jnp.float32)] * K
    + [pltpu.VMEM((16, W), jnp.float32)] * K
    + [pltpu.VMEM((ZR, W), jnp.float32)]
    + [pltpu.VMEM((RPT,), jnp.float32)]
    + [pltpu.VMEM_SHARED((NP, W), jnp.float32)] * 2
    + [pltpu.SemaphoreType.DMA] * (2 * K)
)


@functools.partial(
    pl.kernel,
    mesh=_mesh,
    compiler_params=_sc_params,
    out_type=(
        [jax.ShapeDtypeStruct((NP, W), jnp.float32)] * 8
        + [jax.ShapeDtypeStruct((NP,), jnp.float32)] * 2
    ),
    scratch_types=_SC_SCRATCH + (
        [pltpu.VMEM((CHUNK,), jnp.float32)] * K
        + [pltpu.VMEM((CHUNK,), jnp.float32)]
        + [pltpu.VMEM((RPT,), jnp.float32)]
        + [pltpu.VMEM_SHARED((NP,), jnp.float32)] * 2
        + [pltpu.SemaphoreType.DMA] * (2 * K)
    ),
)
def _gate_conv(xw0, xw1, xw2, xw3, xw4, xw5, xw6, xw7, nidx_h, eidx_h, hw_h,
               o0, o1, o2, o3, o4, o5, o6, o7, dinv_h, binv_h, *scr):
    cid = lax.axis_index("c")
    sid = lax.axis_index("s")
    base = sid * RPT
    it = iter(scr)
    nidx, eidx = next(it), next(it)
    rbufs = [next(it) for _ in range(K)]
    sbufs = [next(it) for _ in range(K)]
    zbuf, bl = next(it), next(it)
    ef_sh, out_sh = next(it), next(it)
    gsems = [next(it) for _ in range(K)]
    ssems = [next(it) for _ in range(K)]
    hwbufs = [next(it) for _ in range(K)]
    ones, dl = next(it), next(it)
    d_sh, bd_sh = next(it), next(it)
    bsems = [next(it) for _ in range(K)]
    csems = [next(it) for _ in range(K)]

    pltpu.sync_copy(nidx_h.at[sid], nidx)
    pltpu.sync_copy(eidx_h.at[sid], eidx)

    _fill_zbuf(zbuf)
    one = jnp.ones((16,), jnp.float32)
    for i in range(CHUNK // 16):
        ones[pl.ds(i * 16, 16)] = one

    _zero_vec(bl, RPT)
    pltpu.sync_copy(bl, d_sh.at[pl.ds(base, RPT)])
    pltpu.sync_copy(bl, bd_sh.at[pl.ds(base, RPT)])

    hist_args = (hw_h, hwbufs, ones, d_sh, bd_sh, bsems, csems)

    def finish_hist():
        pltpu.sync_copy(bd_sh.at[pl.ds(base, RPT)], bl)
        pltpu.sync_copy(d_sh.at[pl.ds(base, RPT)], dl)

        def recip(i, _):
            bd = bl[pl.ds(i * 16, 16)]
            bl[pl.ds(i * 16, 16)] = jnp.where(bd > 0.0, 1.0 / bd, 0.0)
            dv = dl[pl.ds(i * 16, 16)]
            dl[pl.ds(i * 16, 16)] = jnp.where(dv > 0.0, 1.0 / dv, 0.0)
            return 0

        lax.fori_loop(0, RPT // 16, recip, 0)

        @pl.when(cid == 0)
        def _():
            pltpu.sync_copy(bl, binv_h.at[pl.ds(base, RPT)])
            pltpu.sync_copy(dl, dinv_h.at[pl.ds(base, RPT)])

    lo = [(xw0, o0), (xw1, o1), (xw2, o2), (xw3, o3)]
    hi = [(xw4, o4), (xw5, o5), (xw6, o6), (xw7, o7)]
    for s in range(4):
        h = hist_args if s == 0 else None
        fh = finish_hist if s == 0 else None

        @pl.when(cid == 0)
        def _():
            _conv_slice(nidx, eidx, lo[s][0], lo[s][1], base, bl, sbufs,
                        zbuf, rbufs, gsems, ssems, ef_sh, out_sh,
                        hist=h, after_stage1=fh)

        @pl.when(cid == 1)
        def _():
            _conv_slice(nidx, eidx, hi[s][0], hi[s][1], base, bl, sbufs,
                        zbuf, rbufs, gsems, ssems, ef_sh, out_sh,
                        hist=h, after_stage1=fh)


@functools.partial(
    pl.kernel,
    mesh=_mesh,
    compiler_params=_sc_params,
    out_type=[jax.ShapeDtypeStruct((NP, W), jnp.float32)] * 4,
    scratch_types=_SC_SCRATCH,
)
def _cand_conv(c0, c1, c2, c3, nidx_h, eidx_h, binv_h,
               oc0, oc1, oc2, oc3, *scr):
    cid = lax.axis_index("c")
    sid = lax.axis_index("s")
    base = sid * RPT
    it = iter(scr)
    nidx, eidx = next(it), next(it)
    rbufs = [next(it) for _ in range(K)]
    sbufs = [next(it) for _ in range(K)]
    zbuf, bl = next(it), next(it)
    ef_sh, out_sh = next(it), next(it)
    gsems = [next(it) for _ in range(K)]
    ssems = [next(it) for _ in range(K)]

    pltpu.sync_copy(nidx_h.at[sid], nidx)
    pltpu.sync_copy(eidx_h.at[sid], eidx)
    pltpu.sync_copy(binv_h.at[pl.ds(base, RPT)], bl)
    _fill_zbuf(zbuf)

    lo = [(c0, oc0), (c1, oc1)]
    hi = [(c2, oc2), (c3, oc3)]
    for s in range(2):
        @pl.when(cid == 0)
        def _():
            _conv_slice(nidx, eidx, lo[s][0], lo[s][1], base, bl, sbufs,
                        zbuf, rbufs, gsems, ssems, ef_sh, out_sh)

        @pl.when(cid == 1)
        def _():
            _conv_slice(nidx, eidx, hi[s][0], hi[s][1], base, bl, sbufs,
                        zbuf, rbufs, gsems, ssems, ef_sh, out_sh)


_BR = 1024


def _tc1_body(x_ref, st_ref, wt_ref, *outs):
    cat = jnp.concatenate([x_ref[...], st_ref[...]], axis=1)
    xw = jnp.dot(cat, wt_ref[...], preferred_element_type=jnp.float32)
    for k, o in enumerate(outs):
        o[...] = xw[:, k * W:(k + 1) * W]


def _layernorm(a, g, b):
    mu = jnp.mean(a, axis=-1, keepdims=True)
    var = jnp.mean((a - mu) ** 2, axis=-1, keepdims=True)
    return (a - mu) / jnp.sqrt(var + 1e-5) * g + b


def _tc2_body(x_ref, st_ref, o0, o1, o2, o3, o4, o5, o6, o7, dinv_ref,
              gb, gg, gbe, crwt_ref, rb, z_o, res_o, c0_o, c1_o, c2_o, c3_o):
    x = x_ref[...]
    st = st_ref[...]
    cat = jnp.concatenate([x, st], axis=1)
    conv = jnp.concatenate(
        [o[...] for o in (o0, o1, o2, o3, o4, o5, o6, o7)], axis=1)
    conv = conv * dinv_ref[...] + gb[...]
    g = _layernorm(jax.nn.relu(cat + conv), gg[...], gbe[...])
    zr = jax.nn.sigmoid(g)
    z = zr[:, 0:HID]
    r = zr[:, HID:]
    ci = jnp.concatenate([x, r * st], axis=1)
    both = jnp.dot(ci, crwt_ref[...], preferred_element_type=jnp.float32)
    z_o[...] = z
    res_o[...] = both[:, HID:] + rb[...]
    for k, o in enumerate((c0_o, c1_o, c2_o, c3_o)):
        o[...] = both[:, k * W:(k + 1) * W]


def _tc3_body(z_ref, res_ref, st_ref, oc0, oc1, oc2, oc3, dinv_ref,
              cb, cg, cbe, h_o):
    conv = jnp.concatenate([o[...] for o in (oc0, oc1, oc2, oc3)], axis=1)
    conv = conv * dinv_ref[...] + cb[...]
    c = _layernorm(jax.nn.relu(res_ref[...] + conv), cg[...], cbe[...])
    hc = jnp.tanh(c)
    z = z_ref[...]
    h_o[...] = (1.0 - z) * st_ref[...] + z * hc


def _row_spec(w):
    return pl.BlockSpec((_BR, w), lambda i: (i, 0))


def _full_spec(shape):
    return pl.BlockSpec(shape, lambda i: tuple(0 for _ in shape))


def kernel(x, state, hyperedge_index, hyperedge_weight, gate_W, gate_b,
           gate_g, gate_beta, cand_W, cand_b, cand_g, cand_beta, res_W,
           res_b):
    f32 = jnp.float32
    node = hyperedge_index[0]
    edge = hyperedge_index[1]
    pad = PAD_BASE + (jnp.arange(NNZ_PAD - NNZ, dtype=jnp.int32) % (NP - PAD_BASE))
    node_r = jnp.concatenate([node, pad]).reshape(TPS, NCH, CHUNK)
    edge_r = jnp.concatenate([edge, pad]).reshape(TPS, NCH, CHUNK)
    hwp = jnp.concatenate(
        [hyperedge_weight, jnp.zeros((NP - hyperedge_weight.shape[0],), f32)])

    gwt = gate_W.T
    xws = pl.pallas_call(
        _tc1_body,
        grid=(NP // _BR,),
        in_specs=[_row_spec(IN_F), _row_spec(HID), _full_spec((256, 256))],
        out_specs=[_row_spec(W)] * 8,
        out_shape=[jax.ShapeDtypeStruct((NP, W), f32)] * 8,
    )(x, state, gwt)

    *os_, dinv, binv = _gate_conv(*xws, node_r, edge_r, hwp)
    dinv2 = dinv.reshape(NP, 1)

    crwt = jnp.concatenate([cand_W, res_W], axis=0).T
    z, res, c0, c1, c2, c3 = pl.pallas_call(
        _tc2_body,
        grid=(NP // _BR,),
        in_specs=[_row_spec(IN_F), _row_spec(HID)] + [_row_spec(W)] * 8 +
                 [_row_spec(1), _full_spec((1, 256)), _full_spec((1, 256)),
                  _full_spec((1, 256)), _full_spec((256, 256)),
                  _full_spec((1, HID))],
        out_specs=[_row_spec(HID), _row_spec(HID)] + [_row_spec(W)] * 4,
        out_shape=[jax.ShapeDtypeStruct((N_NODES, HID), f32),
                   jax.ShapeDtypeStruct((N_NODES, HID), f32)] +
                  [jax.ShapeDtypeStruct((NP, W), f32)] * 4,
    )(x, state, *os_, dinv2, gate_b.reshape(1, 256),
      gate_g.reshape(1, 256), gate_beta.reshape(1, 256), crwt,
      res_b.reshape(1, HID))

    ocs = _cand_conv(c0, c1, c2, c3, node_r, edge_r, binv)

    h = pl.pallas_call(
        _tc3_body,
        grid=(NP // _BR,),
        in_specs=[_row_spec(HID), _row_spec(HID), _row_spec(HID)] +
                 [_row_spec(W)] * 4 +
                 [_row_spec(1), _full_spec((1, HID)), _full_spec((1, HID)),
                  _full_spec((1, HID))],
        out_specs=_row_spec(HID),
        out_shape=jax.ShapeDtypeStruct((N_NODES, HID), f32),
    )(z, res, state, *ocs, dinv2, cand_b.reshape(1, HID),
      cand_g.reshape(1, HID), cand_beta.reshape(1, HID))
    return h


# speedup vs baseline: 16.1305x; 1.3581x over previous
import functools

import jax
import jax.numpy as jnp
from jax import lax
from jax.experimental import pallas as pl
from jax.experimental.pallas import tpu as pltpu
from jax.experimental.pallas import tpu_sc as plsc

N_NODES = 10000
NNZ = 320000
IN_F = 128
HID = 128

NP = 10240
W = 64
TPS = 16
CHUNK = 128
NNZ_PAD = 327680
NCH = NNZ_PAD // (TPS * CHUNK)
RPT = NP // TPS
PAD_BASE = 10000
K = 4
ZR = 64

_mesh = plsc.VectorSubcoreMesh(core_axis_name="c", subcore_axis_name="s")
_sc_params = pltpu.CompilerParams(use_tc_tiling_on_sc=False)


def _zero_vec(ref, n):
    z = jnp.zeros((16,), jnp.float32)

    def body(i, _):
        ref[pl.ds(i * 16, 16)] = z
        return 0

    lax.fori_loop(0, n // 16, body, 0)


def _fill_zbuf(zbuf):
    z = jnp.zeros((16,), jnp.float32)
    for r in range(ZR):
        for c in range(W // 16):
            zbuf[r, pl.ds(c * 16, 16)] = z


def _zero_acc(zbuf, acc_sh, base, sem):
    n = RPT // ZR

    def fire(b, _):
        pltpu.async_copy(zbuf, acc_sh.at[pl.ds(base + b * ZR, ZR), :], sem)
        return 0

    lax.fori_loop(0, n, fire, 0)

    def drain(b, _):
        pltpu.make_async_copy(
            zbuf, acc_sh.at[pl.ds(base + b * ZR, ZR), :], sem).wait()
        return 0

    lax.fori_loop(0, n, drain, 0)


def _pair_sweep(nidx, eidx, src, dst, rbufs, gsems, ssems, hist=None):
    if hist is not None:
        hw_h, hwbufs, ones, d_sh, bd_sh, bsems, csems = hist

    for i in range(K):
        pltpu.async_copy(src.at[nidx.at[i]], rbufs[i], gsems[i])
        if hist is not None:
            pltpu.async_copy(hw_h.at[eidx.at[i]], hwbufs[i], csems[i])

    def body(g, _):
        j = g * K
        for i in range(K):
            pltpu.make_async_copy(
                src.at[nidx.at[j + i]], rbufs[i], gsems[i]).wait()
            pltpu.async_copy(
                rbufs[i], dst.at[eidx.at[j + i]], ssems[i], add=True)
            if hist is not None:
                pltpu.make_async_copy(
                    hw_h.at[eidx.at[j + i]], hwbufs[i], csems[i]).wait()
                pltpu.async_copy(
                    hwbufs[i], d_sh.at[nidx.at[j + i]], csems[i], add=True)
                pltpu.async_copy(
                    ones, bd_sh.at[eidx.at[j + i]], bsems[i], add=True)
        for i in range(K):
            nj = j + i + K

            @pl.when(nj < NCH)
            def _():
                pltpu.make_async_copy(
                    rbufs[i], dst.at[eidx.at[j + i]], ssems[i]).wait()
                pltpu.async_copy(src.at[nidx.at[nj]], rbufs[i], gsems[i])
                if hist is not None:
                    pltpu.make_async_copy(
                        hwbufs[i], d_sh.at[nidx.at[j + i]], csems[i]).wait()
                    pltpu.make_async_copy(
                        ones, bd_sh.at[eidx.at[j + i]], bsems[i]).wait()
                    pltpu.async_copy(
                        hw_h.at[eidx.at[nj]], hwbufs[i], csems[i])
        return 0

    lax.fori_loop(0, NCH // K, body, 0)
    for i in range(K):
        cj = NCH - K + i
        pltpu.make_async_copy(
            rbufs[i], dst.at[eidx.at[cj]], ssems[i]).wait()
        if hist is not None:
            pltpu.make_async_copy(
                hwbufs[i], d_sh.at[nidx.at[cj]], csems[i]).wait()
            pltpu.make_async_copy(
                ones, bd_sh.at[eidx.at[cj]], bsems[i]).wait()


def _export_clear(acc_sh, base, dst_hbm, bl, sbufs, zbuf, gsems, ssems,
                  tsem):
    nblk = RPT // 16

    def ablk(b):
        return acc_sh.at[pl.ds(base + b * 16, 16), :]

    def dblk(b):
        return dst_hbm.at[pl.ds(base + b * 16, 16), :]

    zb16 = zbuf.at[pl.ds(0, 16), :]

    for i in range(K):
        pltpu.async_copy(ablk(i), sbufs[i], gsems[i])

    def body(g, _):
        j = g * K
        for i in range(K):
            b = j + i
            pltpu.make_async_copy(ablk(b), sbufs[i], gsems[i]).wait()
            if bl is not None:
                bv16 = bl[pl.ds(b * 16, 16)]
                for r in range(16):
                    bvec = lax.broadcast(bv16[r], (16,))
                    for c in range(W // 16):
                        sbufs[i][r, pl.ds(c * 16, 16)] = (
                            sbufs[i][r, pl.ds(c * 16, 16)] * bvec)
            pltpu.async_copy(sbufs[i], dblk(b), ssems[i])
            pltpu.async_copy(zb16, ablk(b), tsem)
        for i in range(K):
            nb = j + i + K

            @pl.when(nb < nblk)
            def _():
                pltpu.make_async_copy(sbufs[i], dblk(j + i), ssems[i]).wait()
                pltpu.async_copy(ablk(nb), sbufs[i], gsems[i])
        return 0

    lax.fori_loop(0, nblk // K, body, 0)
    for i in range(K):
        pltpu.make_async_copy(sbufs[i], dblk(nblk - K + i), ssems[i]).wait()

    def draiz(b, _):
        pltpu.make_async_copy(zb16, ablk(b), tsem).wait()
        return 0

    lax.fori_loop(0, nblk, draiz, 0)


def _conv_slice(nidx, eidx, tbl, es_hbm, out_hbm, base, bl, sbufs, zbuf,
                rbufs, gsems, ssems, tsem, acc_sh,
                hist=None, after_stage1=None):
    _pair_sweep(nidx, eidx, tbl, acc_sh, rbufs, gsems, ssems, hist=hist)
    plsc.subcore_barrier()
    if after_stage1 is not None:
        after_stage1()
    _export_clear(acc_sh, base, es_hbm, bl, sbufs, zbuf, gsems, ssems, tsem)
    plsc.subcore_barrier()
    _pair_sweep(eidx, nidx, es_hbm, acc_sh, rbufs, gsems, ssems)
    plsc.subcore_barrier()
    _export_clear(acc_sh, base, out_hbm, None, sbufs, zbuf, gsems, ssems,
                  tsem)
    plsc.subcore_barrier()


_SC_SCRATCH = (
    [pltpu.VMEM((NCH, CHUNK), jnp.int32)] * 2
    + [pltpu.VMEM((CHUNK, W), jnp.float32)] * K
    + [pltpu.VMEM((16, W), jnp.float32)] * K
    + [pltpu.VMEM((ZR, W), jnp.float32)]
    + [pltpu.VMEM((RPT,), jnp.float32)]
    + [pltpu.VMEM_SHARED((NP, W), jnp.float32)]
    + [pltpu.SemaphoreType.DMA] * (2 * K + 1)
)


@functools.partial(
    pl.kernel,
    mesh=_mesh,
    compiler_params=_sc_params,
    out_type=(
        [jax.ShapeDtypeStruct((NP, W), jnp.float32)] * 4
        + [jax.ShapeDtypeStruct((NP,), jnp.float32)] * 2
        + [jax.ShapeDtypeStruct((NP, W), jnp.float32)] * 2
    ),
    scratch_types=_SC_SCRATCH + (
        [pltpu.VMEM((CHUNK,), jnp.float32)] * K
        + [pltpu.VMEM((CHUNK,), jnp.float32)]
        + [pltpu.VMEM((RPT,), jnp.float32)]
        + [pltpu.VMEM_SHARED((NP,), jnp.float32)] * 2
        + [pltpu.SemaphoreType.DMA] * (2 * K)
    ),
)
def _gate_conv(xw0, xw1, xw2, xw3, nidx_h, eidx_h, hw_h,
               o0, o1, o2, o3, dinv_h, binv_h, es0, es1, *scr):
    cid = lax.axis_index("c")
    sid = lax.axis_index("s")
    base = sid * RPT
    it = iter(scr)
    nidx, eidx = next(it), next(it)
    rbufs = [next(it) for _ in range(K)]
    sbufs = [next(it) for _ in range(K)]
    zbuf, bl = next(it), next(it)
    acc_sh = next(it)
    gsems = [next(it) for _ in range(K)]
    ssems = [next(it) for _ in range(K)]
    tsem = next(it)
    hwbufs = [next(it) for _ in range(K)]
    ones, dl = next(it), next(it)
    d_sh, bd_sh = next(it), next(it)
    bsems = [next(it) for _ in range(K)]
    csems = [next(it) for _ in range(K)]

    pltpu.sync_copy(nidx_h.at[sid], nidx)
    pltpu.sync_copy(eidx_h.at[sid], eidx)

    _fill_zbuf(zbuf)
    one = jnp.ones((16,), jnp.float32)
    for i in range(CHUNK // 16):
        ones[pl.ds(i * 16, 16)] = one

    _zero_vec(bl, RPT)
    pltpu.sync_copy(bl, d_sh.at[pl.ds(base, RPT)])
    pltpu.sync_copy(bl, bd_sh.at[pl.ds(base, RPT)])
    _zero_acc(zbuf, acc_sh, base, tsem)
    plsc.subcore_barrier()

    hist_args = (hw_h, hwbufs, ones, d_sh, bd_sh, bsems, csems)

    def finish_hist():
        pltpu.sync_copy(bd_sh.at[pl.ds(base, RPT)], bl)
        pltpu.sync_copy(d_sh.at[pl.ds(base, RPT)], dl)

        def recip(i, _):
            bd = bl[pl.ds(i * 16, 16)]
            bl[pl.ds(i * 16, 16)] = jnp.where(bd > 0.0, 1.0 / bd, 0.0)
            dv = dl[pl.ds(i * 16, 16)]
            dl[pl.ds(i * 16, 16)] = jnp.where(dv > 0.0, 1.0 / dv, 0.0)
            return 0

        lax.fori_loop(0, RPT // 16, recip, 0)

        @pl.when(cid == 0)
        def _():
            pltpu.sync_copy(bl, binv_h.at[pl.ds(base, RPT)])
            pltpu.sync_copy(dl, dinv_h.at[pl.ds(base, RPT)])

    lo = [(xw0, o0), (xw1, o1)]
    hi = [(xw2, o2), (xw3, o3)]
    for s in range(2):
        h = hist_args if s == 0 else None
        fh = finish_hist if s == 0 else None

        @pl.when(cid == 0)
        def _():
            _conv_slice(nidx, eidx, lo[s][0], es0, lo[s][1], base, bl,
                        sbufs, zbuf, rbufs, gsems, ssems, tsem, acc_sh,
                        hist=h, after_stage1=fh)

        @pl.when(cid == 1)
        def _():
            _conv_slice(nidx, eidx, hi[s][0], es1, hi[s][1], base, bl,
                        sbufs, zbuf, rbufs, gsems, ssems, tsem, acc_sh,
                        hist=h, after_stage1=fh)


@functools.partial(
    pl.kernel,
    mesh=_mesh,
    compiler_params=_sc_params,
    out_type=(
        [jax.ShapeDtypeStruct((NP, W), jnp.float32)] * 2
        + [jax.ShapeDtypeStruct((NP, W), jnp.float32)] * 2
    ),
    scratch_types=_SC_SCRATCH,
)
def _cand_conv(c0, c1, nidx_h, eidx_h, binv_h,
               oc0, oc1, es0, es1, *scr):
    cid = lax.axis_index("c")
    sid = lax.axis_index("s")
    base = sid * RPT
    it = iter(scr)
    nidx, eidx = next(it), next(it)
    rbufs = [next(it) for _ in range(K)]
    sbufs = [next(it) for _ in range(K)]
    zbuf, bl = next(it), next(it)
    acc_sh = next(it)
    gsems = [next(it) for _ in range(K)]
    ssems = [next(it) for _ in range(K)]
    tsem = next(it)

    pltpu.sync_copy(nidx_h.at[sid], nidx)
    pltpu.sync_copy(eidx_h.at[sid], eidx)
    pltpu.sync_copy(binv_h.at[pl.ds(base, RPT)], bl)
    _fill_zbuf(zbuf)
    _zero_acc(zbuf, acc_sh, base, tsem)
    plsc.subcore_barrier()

    @pl.when(cid == 0)
    def _():
        _conv_slice(nidx, eidx, c0, es0, oc0, base, bl, sbufs, zbuf,
                    rbufs, gsems, ssems, tsem, acc_sh)

    @pl.when(cid == 1)
    def _():
        _conv_slice(nidx, eidx, c1, es1, oc1, base, bl, sbufs, zbuf,
                    rbufs, gsems, ssems, tsem, acc_sh)


_BR = 1024


def _tc1_body(x_ref, st_ref, wt_ref, *outs):
    cat = jnp.concatenate([x_ref[...], st_ref[...]], axis=1)
    xw = jnp.dot(cat, wt_ref[...], preferred_element_type=jnp.float32)
    for k, o in enumerate(outs):
        o[...] = xw[:, k * W:(k + 1) * W]


def _layernorm(a, g, b):
    mu = jnp.mean(a, axis=-1, keepdims=True)
    var = jnp.mean((a - mu) ** 2, axis=-1, keepdims=True)
    return (a - mu) / jnp.sqrt(var + 1e-5) * g + b


def _tc2_body(x_ref, st_ref, o0, o1, o2, o3, dinv_ref,
              gb, gg, gbe, crwt_ref, rb, z_o, res_o, c0_o, c1_o):
    x = x_ref[...]
    st = st_ref[...]
    cat = jnp.concatenate([x, st], axis=1)
    conv = jnp.concatenate([o[...] for o in (o0, o1, o2, o3)], axis=1)
    conv = conv * dinv_ref[...] + gb[...]
    g = _layernorm(jax.nn.relu(cat + conv), gg[...], gbe[...])
    zr = jax.nn.sigmoid(g)
    z = zr[:, 0:HID]
    r = zr[:, HID:]
    ci = jnp.concatenate([x, r * st], axis=1)
    both = jnp.dot(ci, crwt_ref[...], preferred_element_type=jnp.float32)
    z_o[...] = z
    res_o[...] = both[:, HID:] + rb[...]
    c0_o[...] = both[:, 0:W]
    c1_o[...] = both[:, W:HID]


def _tc3_body(z_ref, res_ref, st_ref, oc0, oc1, dinv_ref,
              cb, cg, cbe, h_o):
    conv = jnp.concatenate([oc0[...], oc1[...]], axis=1)
    conv = conv * dinv_ref[...] + cb[...]
    c = _layernorm(jax.nn.relu(res_ref[...] + conv), cg[...], cbe[...])
    hc = jnp.tanh(c)
    z = z_ref[...]
    h_o[...] = (1.0 - z) * st_ref[...] + z * hc


def _row_spec(w):
    return pl.BlockSpec((_BR, w), lambda i: (i, 0))


def _full_spec(shape):
    return pl.BlockSpec(shape, lambda i: tuple(0 for _ in shape))


def kernel(x, state, hyperedge_index, hyperedge_weight, gate_W, gate_b,
           gate_g, gate_beta, cand_W, cand_b, cand_g, cand_beta, res_W,
           res_b):
    f32 = jnp.float32
    node = hyperedge_index[0]
    edge = hyperedge_index[1]
    pad = PAD_BASE + (jnp.arange(NNZ_PAD - NNZ, dtype=jnp.int32)
                      % (NP - PAD_BASE))
    node_r = jnp.concatenate([node, pad]).reshape(TPS, NCH, CHUNK)
    edge_r = jnp.concatenate([edge, pad]).reshape(TPS, NCH, CHUNK)
    hwp = jnp.concatenate(
        [hyperedge_weight, jnp.zeros((NP - hyperedge_weight.shape[0],), f32)])

    gwt = gate_W.T
    xws = pl.pallas_call(
        _tc1_body,
        grid=(NP // _BR,),
        in_specs=[_row_spec(IN_F), _row_spec(HID), _full_spec((256, 256))],
        out_specs=[_row_spec(W)] * 4,
        out_shape=[jax.ShapeDtypeStruct((NP, W), f32)] * 4,
    )(x, state, gwt)

    o0, o1, o2, o3, dinv, binv, _, _ = _gate_conv(*xws, node_r, edge_r, hwp)
    dinv2 = dinv.reshape(NP, 1)

    crwt = jnp.concatenate([cand_W, res_W], axis=0).T
    z, res, c0, c1 = pl.pallas_call(
        _tc2_body,
        grid=(NP // _BR,),
        in_specs=[_row_spec(IN_F), _row_spec(HID)] + [_row_spec(W)] * 4 +
                 [_row_spec(1), _full_spec((1, 256)), _full_spec((1, 256)),
                  _full_spec((1, 256)), _full_spec((256, 256)),
                  _full_spec((1, HID))],
        out_specs=[_row_spec(HID), _row_spec(HID)] + [_row_spec(W)] * 2,
        out_shape=[jax.ShapeDtypeStruct((N_NODES, HID), f32),
                   jax.ShapeDtypeStruct((N_NODES, HID), f32)] +
                  [jax.ShapeDtypeStruct((NP, W), f32)] * 2,
    )(x, state, o0, o1, o2, o3, dinv2, gate_b.reshape(1, 256),
      gate_g.reshape(1, 256), gate_beta.reshape(1, 256), crwt,
      res_b.reshape(1, HID))

    oc0, oc1, _, _ = _cand_conv(c0, c1, node_r, edge_r, binv)

    h = pl.pallas_call(
        _tc3_body,
        grid=(NP // _BR,),
        in_specs=[_row_spec(HID), _row_spec(HID), _row_spec(HID)] +
                 [_row_spec(W)] * 2 +
                 [_row_spec(1), _full_spec((1, HID)), _full_spec((1, HID)),
                  _full_spec((1, HID))],
        out_specs=_row_spec(HID),
        out_shape=jax.ShapeDtypeStruct((N_NODES, HID), f32),
    )(z, res, state, oc0, oc1, dinv2, cand_b.reshape(1, HID),
      cand_g.reshape(1, HID), cand_beta.reshape(1, HID))
    return h
